# baseline (speedup 1.0000x reference)
import functools

import jax
import jax.numpy as jnp
from jax import lax
from jax.experimental import pallas as pl
from jax.experimental.pallas import tpu as pltpu
from jax.experimental.pallas import tpu_sc as plsc

_B = 1024
_D_NUM = 256
_F = 16
_D_TOK = 32
_N_CAT = 26
_CAT_SIZE = 100000

_ROWS = _N_CAT * _D_NUM
_R_PER_W = _ROWS // 32


def _sc_gather_body(idx_hbm, emb_hbm, out_hbm, idx_v, sem):
    w = lax.axis_index("s") * 2 + lax.axis_index("c")
    base = w * _R_PER_W
    pltpu.sync_copy(idx_hbm.at[pl.ds(2 * w, 2)], idx_v)
    lanes = lax.iota(jnp.int32, 16)
    zeros = jnp.zeros((16,), jnp.int32)
    for j in range(_R_PER_W):
        h, l = divmod(j, 128)
        v16 = idx_v[h, pl.ds((l // 16) * 16, 16)]
        r = lax.reduce_sum_p.bind(
            jnp.where(lanes == l % 16, v16, zeros), axes=(0,)
        )
        pltpu.make_async_copy(emb_hbm.at[r], out_hbm.at[base + j], sem).start()
    pltpu.make_async_copy(
        emb_hbm.at[pl.ds(0, _R_PER_W)], out_hbm.at[pl.ds(base, _R_PER_W)], sem
    ).wait()


@functools.cache
def _make_sc_gather():
    mesh = plsc.VectorSubcoreMesh(
        core_axis_name="c", subcore_axis_name="s", num_cores=2, num_subcores=16
    )
    return pl.kernel(
        _sc_gather_body,
        out_type=jax.ShapeDtypeStruct((_ROWS, _D_TOK), jnp.float32),
        mesh=mesh,
        scratch_types=[
            pltpu.VMEM((2, 128), jnp.int32),
            pltpu.SemaphoreType.DMA,
        ],
        compiler_params=pltpu.CompilerParams(needs_layout_passes=False),
    )


_TILE = 128
_GRID = 9


def _tc_body(xt_ref, g_ref, wt_ref, wqt_ref, bt_ref, out_ref):
    i = pl.program_id(0)

    def bdim(src, shape, dims):
        return lax.broadcast_in_dim(src, shape, dims)

    @pl.when(i < _GRID - 1)
    def _dense():
        out_ref[0:_F, :, :] = bdim(wt_ref[0:_F], (_F, _D_TOK, _TILE), (0, 1))
        out_ref[_F:, :, :] = (
            bdim(xt_ref[...], (_D_NUM, _D_TOK, _TILE), (0, 2))
            * bdim(wt_ref[_F:], (_D_NUM, _D_TOK, _TILE), (0, 1))
            + bdim(bt_ref[...], (_D_NUM, _D_TOK, _TILE), (0, 1))
        )

    @pl.when(i == _GRID - 1)
    def _cat():
        out_ref[0:_F, :, 0:_N_CAT] = bdim(wqt_ref[...], (_F, _D_TOK, _N_CAT), (0, 1))
        out_ref[_F:, :, 0:_N_CAT] = g_ref[...] + bdim(
            bt_ref[...], (_D_NUM, _D_TOK, _N_CAT), (0, 1)
        )


def kernel(x_num, x_cat, emb, weight_, weight, bias):
    offsets = jnp.arange(_N_CAT, dtype=jnp.int32) * _CAT_SIZE
    flat = (x_cat + offsets[:, None]).reshape(-1)
    chunks = flat.reshape(32, _R_PER_W)
    idx_pack = jnp.pad(chunks, ((0, 0), (0, 256 - _R_PER_W))).reshape(64, 128)
    g = _make_sc_gather()(idx_pack, emb)
    g3 = g.reshape(_N_CAT, _D_NUM, _D_TOK).transpose(1, 2, 0)
    out_t = pl.pallas_call(
        _tc_body,
        grid=(_GRID,),
        in_specs=[
            pl.BlockSpec((_D_NUM, _TILE), lambda i: (0, jnp.minimum(i, _GRID - 2))),
            pl.BlockSpec((_D_NUM, _D_TOK, _N_CAT), lambda i: (0, 0, 0)),
            pl.BlockSpec((_F + _D_NUM, _D_TOK), lambda i: (0, 0)),
            pl.BlockSpec((_F, _D_TOK), lambda i: (0, 0)),
            pl.BlockSpec((_D_NUM, _D_TOK), lambda i: (0, 0)),
        ],
        out_specs=pl.BlockSpec((_F + _D_NUM, _D_TOK, _TILE), lambda i: (0, 0, i)),
        out_shape=jax.ShapeDtypeStruct((_F + _D_NUM, _D_TOK, _B + _N_CAT), jnp.float32),
    )(x_num.T, g3, weight, weight_, bias)
    return jnp.transpose(out_t, (2, 0, 1))
